# baseline (speedup 1.0000x reference)
import functools

import jax
import jax.numpy as jnp
from jax.experimental import pallas as pl
from jax.experimental.pallas import tpu as pltpu

K_TOP = 2


def _router_kernel(x_ref, wg_ref, wr_ref, w_out_ref, acc_ref, *, nlt, inv_l):
    lt = pl.program_id(0)

    @pl.when(lt == 0)
    def _():
        acc_ref[...] = jnp.zeros_like(acc_ref)

    acc_ref[...] += jnp.sum(x_ref[...], axis=1)

    @pl.when(lt == nlt - 1)
    def _():
        pooled = acc_ref[...] * inv_l
        gated = jax.lax.dot_general(
            pooled, wg_ref[...], (((1,), (1,)), ((), ())),
            preferred_element_type=jnp.float32)
        logits = jax.lax.dot_general(
            gated, wr_ref[...], (((1,), (1,)), ((), ())),
            preferred_element_type=jnp.float32)
        m = jnp.max(logits, axis=-1, keepdims=True)
        p = jnp.exp(logits - m)
        probs = p / jnp.sum(p, axis=-1, keepdims=True)
        e_ids = jax.lax.broadcasted_iota(jnp.int32, probs.shape, 1)
        v1 = jnp.max(probs, axis=-1, keepdims=True)
        i1 = jnp.argmax(probs, axis=-1)[:, None]
        masked = jnp.where(e_ids == i1, -jnp.inf, probs)
        v2 = jnp.max(masked, axis=-1, keepdims=True)
        i2 = jnp.argmax(masked, axis=-1)[:, None]
        denom = v1 + v2
        w = jnp.where(e_ids == i1, v1 / denom, 0.0)
        w = jnp.where(e_ids == i2, v2 / denom, w)
        w_out_ref[...] = w.astype(w_out_ref.dtype)


def _combine_kernel(w_ref, a_ref, b_ref, wbt_ref, x_ref, out_ref, mt_ref, *,
                    r):
    b = pl.program_id(0)
    lt = pl.program_id(1)

    @pl.when(lt == 0)
    def _():
        w = w_ref[b, :]
        e = w.shape[0]
        k_exp = jax.lax.broadcasted_iota(jnp.int32, (e * r, e), 0) // r
        e_ids = jax.lax.broadcasted_iota(jnp.int32, (e * r, e), 1)
        sel = (k_exp == e_ids).astype(jnp.float32)
        w_rep = jnp.sum(sel * w[None, :], axis=1, keepdims=True)
        a_w = a_ref[...] * w_rep
        delta = jax.lax.dot_general(
            a_w, b_ref[...], (((0,), (0,)), ((), ())),
            preferred_element_type=jnp.float32)
        mt_ref[...] = (wbt_ref[...] + delta).astype(mt_ref.dtype)

    xt = x_ref[0].astype(jnp.bfloat16)
    out_ref[0] = jnp.dot(xt, mt_ref[...],
                         preferred_element_type=jnp.float32)


@jax.jit
def kernel(x, W_base, W_g, W_r, lora_A, lora_B):
    B, L, D = x.shape
    E, R, _ = lora_A.shape

    A_cat = lora_A.reshape(E * R, D)
    B_cat = lora_B.transpose(0, 2, 1).reshape(E * R, D)
    Wb_t = W_base.T

    TL_R = 2048
    nlt_r = L // TL_R
    weights = pl.pallas_call(
        functools.partial(_router_kernel, nlt=nlt_r, inv_l=1.0 / L),
        grid=(nlt_r,),
        in_specs=[
            pl.BlockSpec((B, TL_R, D), lambda lt: (0, lt, 0)),
            pl.BlockSpec((D, D), lambda lt: (0, 0)),
            pl.BlockSpec((E, D), lambda lt: (0, 0)),
        ],
        out_specs=pl.BlockSpec((B, E), lambda lt: (0, 0)),
        out_shape=jax.ShapeDtypeStruct((B, E), jnp.float32),
        scratch_shapes=[pltpu.VMEM((B, D), jnp.float32)],
    )(x, W_g, W_r)

    TL = 2048
    nlt = L // TL
    out = pl.pallas_call(
        functools.partial(_combine_kernel, r=R),
        grid=(B, nlt),
        in_specs=[
            pl.BlockSpec((B, E), lambda b, lt: (0, 0)),
            pl.BlockSpec((E * R, D), lambda b, lt: (0, 0)),
            pl.BlockSpec((E * R, D), lambda b, lt: (0, 0)),
            pl.BlockSpec((D, D), lambda b, lt: (0, 0)),
            pl.BlockSpec((1, TL, D), lambda b, lt: (b, lt, 0)),
        ],
        out_specs=pl.BlockSpec((1, TL, D), lambda b, lt: (b, lt, 0)),
        out_shape=jax.ShapeDtypeStruct((B, L, D), jnp.float32),
        scratch_shapes=[pltpu.VMEM((D, D), jnp.bfloat16)],
    )(weights, A_cat, B_cat, Wb_t, x)

    return out


# speedup vs baseline: 1.0456x; 1.0456x over previous
import functools

import jax
import jax.numpy as jnp
from jax.experimental import pallas as pl
from jax.experimental.pallas import tpu as pltpu

K_TOP = 2


def _tile_partial_sum(t):
    tl, d = t.shape
    return jnp.sum(t.reshape(tl // 8, 8, d), axis=0)


def _router_weights(pooled, wg, wr):
    hi = jax.lax.Precision.HIGHEST
    gated = jax.lax.dot_general(pooled, wg, (((1,), (1,)), ((), ())),
                                precision=hi,
                                preferred_element_type=jnp.float32)
    logits = jax.lax.dot_general(gated, wr, (((1,), (1,)), ((), ())),
                                 precision=hi,
                                 preferred_element_type=jnp.float32)
    m = jnp.max(logits, axis=-1, keepdims=True)
    p = jnp.exp(logits - m)
    probs = p / jnp.sum(p, axis=-1, keepdims=True)
    e_ids = jax.lax.broadcasted_iota(jnp.int32, probs.shape, 1)
    v1 = jnp.max(probs, axis=-1, keepdims=True)
    i1 = jnp.argmax(probs, axis=-1)[:, None]
    masked = jnp.where(e_ids == i1, -jnp.inf, probs)
    v2 = jnp.max(masked, axis=-1, keepdims=True)
    i2 = jnp.argmax(masked, axis=-1)[:, None]
    denom = v1 + v2
    w = jnp.where(e_ids == i1, v1 / denom, 0.0)
    w = jnp.where(e_ids == i2, v2 / denom, w)
    return w


def _fused_kernel(xh_ref, wg_ref, wr_ref, a_ref, bcat_ref, wbt_ref, out_ref,
                  xbuf, acc, mt, sems, *, nb, nlt, tl, r, inv_l):
    bb = pl.program_id(0)
    lt = pl.program_id(1)
    d = wbt_ref.shape[0]

    def tile_copy(batch, tile, par):
        return pltpu.make_async_copy(
            xh_ref.at[batch, pl.ds(tile * tl, tl), :],
            xbuf.at[par, pl.ds(tile * tl, tl), :],
            sems.at[tile],
        )

    par_p = jax.lax.rem(bb, 2)

    @pl.when(bb < nb)
    def _():
        tile_copy(bb, lt, par_p).start()

    @pl.when((bb < nb) & (lt >= 1))
    def _():
        tile_copy(bb, lt - 1, par_p).wait()
        part = _tile_partial_sum(xbuf[par_p, pl.ds((lt - 1) * tl, tl), :])

        @pl.when(lt == 1)
        def _():
            acc[par_p] = part

        @pl.when(lt > 1)
        def _():
            acc[par_p] += part

    cb = bb - 1
    par_c = jax.lax.rem(bb + 1, 2)

    @pl.when(bb >= 1)
    def _():
        @pl.when(lt == 0)
        def _():
            tile_copy(cb, nlt - 1, par_c).wait()
            last = _tile_partial_sum(xbuf[par_c, pl.ds((nlt - 1) * tl, tl), :])
            pooled = (jnp.sum(acc[par_c] + last, axis=0, keepdims=True)
                      * inv_l)
            w = _router_weights(pooled, wg_ref[...], wr_ref[...])
            e = w.shape[1]
            k_exp = jax.lax.broadcasted_iota(jnp.int32, (e * r, e), 0) // r
            eids = jax.lax.broadcasted_iota(jnp.int32, (e * r, e), 1)
            sel = (k_exp == eids).astype(jnp.float32)
            w_rep = jnp.sum(sel * w, axis=1, keepdims=True)
            a_w = a_ref[...] * w_rep
            delta = jax.lax.dot_general(
                a_w, bcat_ref[...], (((0,), (0,)), ((), ())),
                preferred_element_type=jnp.float32)
            mt[...] = (wbt_ref[...] + delta).astype(mt.dtype)

        xt = xbuf[par_c, pl.ds(lt * tl, tl), :].astype(jnp.bfloat16)
        out_ref[0] = jnp.dot(xt, mt[...],
                             preferred_element_type=jnp.float32)


@jax.jit
def kernel(x, W_base, W_g, W_r, lora_A, lora_B):
    B, L, D = x.shape
    E, R, _ = lora_A.shape

    A_cat = lora_A.reshape(E * R, D)
    B_cat = lora_B.transpose(0, 2, 1).reshape(E * R, D)
    Wb_t = W_base.T

    TL = 1024
    NLT = L // TL

    out = pl.pallas_call(
        functools.partial(_fused_kernel, nb=B, nlt=NLT, tl=TL, r=R,
                          inv_l=1.0 / L),
        grid=(B + 1, NLT),
        in_specs=[
            pl.BlockSpec(memory_space=pltpu.MemorySpace.HBM),
            pl.BlockSpec((D, D), lambda bb, lt: (0, 0)),
            pl.BlockSpec((E, D), lambda bb, lt: (0, 0)),
            pl.BlockSpec((E * R, D), lambda bb, lt: (0, 0)),
            pl.BlockSpec((E * R, D), lambda bb, lt: (0, 0)),
            pl.BlockSpec((D, D), lambda bb, lt: (0, 0)),
        ],
        out_specs=pl.BlockSpec(
            (1, TL, D),
            lambda bb, lt: (jnp.maximum(bb - 1, 0),
                            jnp.where(bb == 0, 0, lt), 0)),
        out_shape=jax.ShapeDtypeStruct((B, L, D), jnp.float32),
        scratch_shapes=[
            pltpu.VMEM((2, L, D), jnp.float32),
            pltpu.VMEM((2, 8, D), jnp.float32),
            pltpu.VMEM((D, D), jnp.bfloat16),
            pltpu.SemaphoreType.DMA((NLT,)),
        ],
        compiler_params=pltpu.CompilerParams(
            vmem_limit_bytes=100 * 1024 * 1024),
    )(x, W_g, W_r, A_cat, B_cat, Wb_t)

    return out
